# baseline (speedup 1.0000x reference)
import jax
import jax.numpy as jnp
from jax.experimental import pallas as pl
from jax.experimental.pallas import tpu as pltpu

B, S, H, E = 4, 4096, 2048, 8
N = B * S
T = 2048
KC = 128
NC = H // KC


def _router_block(x_ref, wt_ref, logits_ref, idx_ref, w_ref, acc_ref):
    c = pl.program_id(1)
    partial = jnp.dot(x_ref[...].astype(jnp.bfloat16),
                      wt_ref[...].astype(jnp.bfloat16),
                      preferred_element_type=jnp.float32)

    @pl.when(c == 0)
    def _():
        acc_ref[...] = partial

    @pl.when(c > 0)
    def _():
        acc_ref[...] = acc_ref[...] + partial

    @pl.when(c == NC - 1)
    def _():
        logits = acc_ref[...]
        logits_ref[...] = logits
        mx = jnp.max(logits, axis=1, keepdims=True)
        iota = jax.lax.broadcasted_iota(jnp.int32, logits.shape, 1)
        idx = jnp.min(jnp.where(logits == mx, iota, E), axis=1, keepdims=True)
        idx_ref[...] = idx
        w_ref[...] = jnp.ones_like(mx)


@jax.jit
def kernel(hidden_states, W):
    x = hidden_states.reshape(N, H)
    wt = W.T
    logits, idx, weights = pl.pallas_call(
        _router_block,
        grid=(N // T, NC),
        in_specs=[
            pl.BlockSpec((T, KC), lambda i, c: (i, c)),
            pl.BlockSpec((KC, E), lambda i, c: (c, 0)),
        ],
        out_specs=[
            pl.BlockSpec((T, E), lambda i, c: (i, 0)),
            pl.BlockSpec((T, 1), lambda i, c: (i, 0)),
            pl.BlockSpec((T, 1), lambda i, c: (i, 0)),
        ],
        out_shape=[
            jax.ShapeDtypeStruct((N, E), jnp.float32),
            jax.ShapeDtypeStruct((N, 1), jnp.int32),
            jax.ShapeDtypeStruct((N, 1), jnp.float32),
        ],
        scratch_shapes=[pltpu.MemorySpace.VMEM((T, E), jnp.float32)],
        compiler_params=pltpu.CompilerParams(
            dimension_semantics=("parallel", "arbitrary"),
        ),
    )(x, wt)
    return (
        logits.reshape(B, S, E),
        idx.reshape(B, S),
        weights.reshape(B, S),
    )


# speedup vs baseline: 2.0215x; 2.0215x over previous
import jax
import jax.numpy as jnp
from jax.experimental import pallas as pl
from jax.experimental.pallas import tpu as pltpu

B, S, H, E = 4, 4096, 2048, 8
N = B * S
T = 2048
KC = 128
NC = H // KC


def _router_block(x_ref, w_ref, logits_ref, idx_ref, wout_ref):
    HM = H // 2
    lm = jnp.dot(x_ref[:, :HM].astype(jnp.bfloat16),
                 w_ref[:, :HM].T.astype(jnp.bfloat16),
                 preferred_element_type=jnp.float32)
    accs = [None] * E
    for c in range(NC // 2, NC):
        xc = x_ref[:, c * KC:(c + 1) * KC]
        for e in range(E):
            t = xc * w_ref[e:e + 1, c * KC:(c + 1) * KC]
            accs[e] = t if accs[e] is None else accs[e] + t
    cols = [jnp.sum(a, axis=1, keepdims=True) for a in accs]
    logits = lm + jnp.concatenate(cols, axis=1)
    logits_ref[...] = logits
    mx = jnp.max(logits, axis=1, keepdims=True)
    iota = jax.lax.broadcasted_iota(jnp.int32, logits.shape, 1)
    idx = jnp.min(jnp.where(logits == mx, iota, E), axis=1, keepdims=True)
    idx_ref[...] = idx
    wout_ref[...] = jnp.ones_like(mx)


@jax.jit
def kernel(hidden_states, W):
    x = hidden_states.reshape(N, H)
    logits, idx, weights = pl.pallas_call(
        _router_block,
        grid=(N // T,),
        in_specs=[
            pl.BlockSpec((T, H), lambda i: (i, 0)),
            pl.BlockSpec((E, H), lambda i: (0, 0)),
        ],
        out_specs=[
            pl.BlockSpec((T, E), lambda i: (i, 0)),
            pl.BlockSpec((T, 1), lambda i: (i, 0)),
            pl.BlockSpec((T, 1), lambda i: (i, 0)),
        ],
        out_shape=[
            jax.ShapeDtypeStruct((N, E), jnp.float32),
            jax.ShapeDtypeStruct((N, 1), jnp.int32),
            jax.ShapeDtypeStruct((N, 1), jnp.float32),
        ],
        compiler_params=pltpu.CompilerParams(
            dimension_semantics=("parallel",),
        ),
    )(x, W)
    return (
        logits.reshape(B, S, E),
        idx.reshape(B, S),
        weights.reshape(B, S),
    )
